# Initial kernel scaffold; baseline (speedup 1.0000x reference)
import functools

import jax
import jax.numpy as jnp
from jax import lax
from jax.experimental import pallas as pl
from jax.experimental.pallas import tpu as pltpu
from jax.experimental.pallas import tpu_sc as plsc

NC = 2
NS = 16
NW = NC * NS

N = 10000
D = 128
NLAYER = 3
E = 320000

NPAD = 10240
NODES_PER_W = NPAD // NW
KA = 80

EPW = 10112
EPAD = NW * EPW
K = 128
NCH = EPW // K

_SC_MESH = plsc.VectorSubcoreMesh(
    core_axis_name="c", subcore_axis_name="s", num_cores=NC, num_subcores=NS
)


@functools.partial(
    pl.kernel,
    out_type=jax.ShapeDtypeStruct((NPAD, D), jnp.float32),
    mesh=_SC_MESH,
    scratch_types=[
        pltpu.VMEM_SHARED((9 * 119, D), jnp.float32),
        pltpu.VMEM((KA,), jnp.int32),
        pltpu.VMEM((9, KA, D), jnp.float32),
        pltpu.VMEM((KA, D), jnp.float32),
        pltpu.SemaphoreType.DMA,
    ],
)
def _atom_kernel(tab_hbm, idx_hbm, out_hbm, tab_sh, idxb, gb, hb, sem):
    c = lax.axis_index("c")
    s = lax.axis_index("s")
    wid = c * NS + s

    @pl.when(s == 0)
    def _():
        pltpu.sync_copy(tab_hbm, tab_sh)

    plsc.subcore_barrier()

    def chunk(i, carry):
        base = wid * NODES_PER_W + i * KA
        for f in range(9):
            pltpu.sync_copy(idx_hbm.at[f, pl.ds(base, KA)], idxb)
            pltpu.async_copy(tab_sh.at[idxb], gb.at[f], sem).wait()

        def rbody(r, carry2):
            for j in range(8):
                sl = pl.ds(j * 16, 16)
                v = gb[0, r, sl]
                for f in range(1, 9):
                    v = v + gb[f, r, sl]
                hb[r, sl] = v
            return carry2

        lax.fori_loop(0, KA, rbody, 0)
        pltpu.sync_copy(hb, out_hbm.at[pl.ds(base, KA)])
        return carry

    lax.fori_loop(0, NODES_PER_W // KA, chunk, 0)


@functools.partial(
    pl.kernel,
    out_type=jax.ShapeDtypeStruct((NC, N, D), jnp.float32),
    mesh=_SC_MESH,
    scratch_types=[
        pltpu.VMEM_SHARED((128, D), jnp.float32),
        pltpu.VMEM_SHARED((N, D), jnp.float32),
        pltpu.VMEM((K,), jnp.int32),
        pltpu.VMEM((K,), jnp.int32),
        pltpu.VMEM((K,), jnp.int32),
        pltpu.VMEM((K,), jnp.float32),
        pltpu.VMEM((K, D), jnp.float32),
        pltpu.VMEM((K, D), jnp.float32),
        pltpu.SemaphoreType.DMA,
        pltpu.SemaphoreType.DMA,
    ],
)
def _agg_kernel(h_hbm, row_hbm, col_hbm, comb_hbm, ew_hbm, ctab_hbm, zer_hbm,
                out_hbm, ctab_sh, acc_sh, rowb, colb, combb, ewb, hbuf, eebuf,
                semh, seme):
    c = lax.axis_index("c")
    s = lax.axis_index("s")
    wid = c * NS + s

    @pl.when(s == 0)
    def _():
        pltpu.sync_copy(ctab_hbm, ctab_sh)

    nz = N // NS
    pltpu.sync_copy(zer_hbm.at[pl.ds(s * nz, nz)], acc_sh.at[pl.ds(s * nz, nz)])
    plsc.subcore_barrier()

    def chunk(i, carry):
        base = wid * EPW + i * K
        pltpu.sync_copy(row_hbm.at[pl.ds(base, K)], rowb)
        pltpu.sync_copy(col_hbm.at[pl.ds(base, K)], colb)
        pltpu.sync_copy(comb_hbm.at[pl.ds(base, K)], combb)
        pltpu.sync_copy(ew_hbm.at[pl.ds(base, K)], ewb)
        cp1 = pltpu.async_copy(h_hbm.at[rowb], hbuf, semh)
        cp2 = pltpu.async_copy(ctab_sh.at[combb], eebuf, seme)
        cp1.wait()
        cp2.wait()

        def ebody(e, carry2):
            w = ewb[e]
            for j in range(8):
                sl = pl.ds(j * 16, 16)
                hbuf[e, sl] = jnp.maximum(hbuf[e, sl] + eebuf[e, sl], 0.0) * w
            return carry2

        lax.fori_loop(0, K, ebody, 0)
        pltpu.sync_copy(hbuf, acc_sh.at[colb], add=True)
        return carry

    lax.fori_loop(0, NCH, chunk, 0)
    plsc.subcore_barrier()
    pltpu.sync_copy(acc_sh.at[pl.ds(s * nz, nz)], out_hbm.at[c, pl.ds(s * nz, nz)])


def _mlp(h, a0, a1, w1, b1, g1, be1, w2, b2, bg, bb, ep, relu_out):
    def body(h_ref, a0_ref, a1_ref, w1_ref, b1_ref, g1_ref, be1_ref, w2_ref,
             b2_ref, bg_ref, bb_ref, ep_ref, o_ref):
        z = (1.0 + ep_ref[0, 0]) * h_ref[...] + a0_ref[...] + a1_ref[...]
        z1 = jnp.dot(z, w1_ref[...], preferred_element_type=jnp.float32)
        z1 = z1 + b1_ref[...]
        m = jnp.mean(z1, axis=0, keepdims=True)
        v = jnp.mean(z1 * z1, axis=0, keepdims=True) - m * m
        z1 = (z1 - m) * lax.rsqrt(v + 1e-5) * g1_ref[...] + be1_ref[...]
        z1 = jnp.maximum(z1, 0.0)
        z2 = jnp.dot(z1, w2_ref[...], preferred_element_type=jnp.float32)
        z2 = z2 + b2_ref[...]
        m2 = jnp.mean(z2, axis=0, keepdims=True)
        v2 = jnp.mean(z2 * z2, axis=0, keepdims=True) - m2 * m2
        z2 = (z2 - m2) * lax.rsqrt(v2 + 1e-5) * bg_ref[...] + bb_ref[...]
        if relu_out:
            z2 = jnp.maximum(z2, 0.0)
        o_ref[...] = z2

    return pl.pallas_call(
        body,
        out_shape=jax.ShapeDtypeStruct((N, D), jnp.float32),
    )(h, a0, a1, w1, b1.reshape(1, -1), g1.reshape(1, -1), be1.reshape(1, -1),
      w2, b2.reshape(1, -1), bg.reshape(1, -1), bb.reshape(1, -1),
      ep.reshape(1, 1))


def kernel(x, edge_index, edge_attr, edge_weight, atom_emb, bond_emb, W1, b1,
           g1, be1, W2, b2, eps, bn_g, bn_b):
    xi = x.astype(jnp.int32)
    idx_atom = (xi + jnp.arange(9, dtype=jnp.int32)[None, :] * 119).T
    padn = jnp.broadcast_to(
        (jnp.arange(NPAD - N, dtype=jnp.int32) % (9 * 119))[None, :],
        (9, NPAD - N))
    idx_atom = jnp.concatenate([idx_atom, padn], axis=1)
    atab = atom_emb.astype(jnp.float32).reshape(9 * 119, D)

    row = edge_index[0].astype(jnp.int32)
    col = edge_index[1].astype(jnp.int32)
    ea = edge_attr.astype(jnp.int32)
    comb = ea[:, 0] * 25 + ea[:, 1] * 5 + ea[:, 2]
    npe = EPAD - E
    pr = jnp.arange(npe, dtype=jnp.int32)
    row_p = jnp.concatenate([row, pr % N])
    col_p = jnp.concatenate([col, pr % N])
    comb_p = jnp.concatenate([comb, pr % 125])
    ew_p = jnp.concatenate(
        [edge_weight.astype(jnp.float32), jnp.zeros((npe,), jnp.float32)])

    ct = (bond_emb[:, 0][:, :, None, None, :]
          + bond_emb[:, 1][:, None, :, None, :]
          + bond_emb[:, 2][:, None, None, :, :]).reshape(NLAYER, 125, D)
    ct = jnp.concatenate(
        [ct, jnp.zeros((NLAYER, 3, D), jnp.float32)], axis=1)
    zer = jnp.zeros((N, D), jnp.float32)

    h = _atom_kernel(atab, idx_atom)[:N]
    for l in range(NLAYER):
        agg = _agg_kernel(h, row_p, col_p, comb_p, ew_p, ct[l], zer)
        h = _mlp(h, agg[0], agg[1], W1[l], b1[l], g1[l], be1[l], W2[l], b2[l],
                 bn_g[l], bn_b[l], eps[l], relu_out=(l < NLAYER - 1))
    return h


# speedup vs baseline: 4.8588x; 4.8588x over previous
import functools

import jax
import jax.numpy as jnp
from jax import lax
from jax.experimental import pallas as pl
from jax.experimental.pallas import tpu as pltpu
from jax.experimental.pallas import tpu_sc as plsc

NC = 2
NS = 16
NW = NC * NS

N = 10000
D = 128
NLAYER = 3
E = 320000

NPAD = 10240
NODES_PER_W = NPAD // NW
KA = 80

EPW = 10112
EPAD = NW * EPW
K = 128
NCH = EPW // K

_SC_MESH = plsc.VectorSubcoreMesh(
    core_axis_name="c", subcore_axis_name="s", num_cores=NC, num_subcores=NS
)


@functools.partial(
    pl.kernel,
    out_type=jax.ShapeDtypeStruct((NPAD, D), jnp.float32),
    mesh=_SC_MESH,
    scratch_types=[
        pltpu.VMEM_SHARED((9 * 119, D), jnp.float32),
        pltpu.VMEM((KA,), jnp.int32),
        pltpu.VMEM((9, KA, D), jnp.float32),
        pltpu.VMEM((KA, D), jnp.float32),
        pltpu.SemaphoreType.DMA,
    ],
)
def _atom_kernel(tab_hbm, idx_hbm, out_hbm, tab_sh, idxb, gb, hb, sem):
    c = lax.axis_index("c")
    s = lax.axis_index("s")
    wid = c * NS + s

    @pl.when(s == 0)
    def _():
        pltpu.sync_copy(tab_hbm, tab_sh)

    plsc.subcore_barrier()

    def chunk(i, carry):
        base = wid * NODES_PER_W + i * KA
        for f in range(9):
            pltpu.sync_copy(idx_hbm.at[pl.ds(f * NPAD + base, KA)], idxb)
            pltpu.async_copy(tab_sh.at[idxb], gb.at[f], sem).wait()

        def rbody(r, carry2):
            for j in range(8):
                sl = pl.ds(j * 16, 16)
                v = gb[0, r, sl]
                for f in range(1, 9):
                    v = v + gb[f, r, sl]
                hb[r, sl] = v
            return carry2

        lax.fori_loop(0, KA, rbody, 0)
        pltpu.sync_copy(hb, out_hbm.at[pl.ds(base, KA)])
        return carry

    lax.fori_loop(0, NODES_PER_W // KA, chunk, 0)


@functools.partial(
    pl.kernel,
    out_type=jax.ShapeDtypeStruct((NC, NPAD, D), jnp.float32),
    mesh=_SC_MESH,
    scratch_types=[
        pltpu.VMEM_SHARED((128, D), jnp.float32),
        pltpu.VMEM_SHARED((NPAD, D), jnp.float32),
        pltpu.VMEM((K,), jnp.int32),
        pltpu.VMEM((K,), jnp.int32),
        pltpu.VMEM((K,), jnp.int32),
        pltpu.VMEM((K,), jnp.float32),
        pltpu.VMEM((K, D), jnp.float32),
        pltpu.VMEM((K, D), jnp.float32),
        pltpu.SemaphoreType.DMA,
        pltpu.SemaphoreType.DMA,
    ],
)
def _agg_kernel(h_hbm, row_hbm, col_hbm, comb_hbm, ew_hbm, ctab_hbm, zer_hbm,
                out_hbm, ctab_sh, acc_sh, rowb, colb, combb, ewb, hbuf, eebuf,
                semh, seme):
    c = lax.axis_index("c")
    s = lax.axis_index("s")
    wid = c * NS + s

    @pl.when(s == 0)
    def _():
        pltpu.sync_copy(ctab_hbm, ctab_sh)

    nz = NPAD // NS
    pltpu.sync_copy(zer_hbm.at[pl.ds(s * nz, nz)], acc_sh.at[pl.ds(s * nz, nz)])
    plsc.subcore_barrier()

    def chunk(i, carry):
        base = wid * EPW + i * K
        pltpu.sync_copy(row_hbm.at[pl.ds(base, K)], rowb)
        pltpu.sync_copy(col_hbm.at[pl.ds(base, K)], colb)
        pltpu.sync_copy(comb_hbm.at[pl.ds(base, K)], combb)
        pltpu.sync_copy(ew_hbm.at[pl.ds(base, K)], ewb)
        cp1 = pltpu.async_copy(h_hbm.at[rowb], hbuf, semh)
        cp2 = pltpu.async_copy(ctab_sh.at[combb], eebuf, seme)
        cp1.wait()
        cp2.wait()

        def gbody(g, carry2):
            wv = ewb[pl.ds(g * 16, 16)]
            for i in range(16):
                e = g * 16 + i
                w = wv[i]
                for j in range(8):
                    sl = pl.ds(j * 16, 16)
                    hbuf[e, sl] = jnp.maximum(
                        hbuf[e, sl] + eebuf[e, sl], 0.0) * w
            return carry2

        lax.fori_loop(0, K // 16, gbody, 0)
        pltpu.sync_copy(hbuf, acc_sh.at[colb], add=True)
        return carry

    lax.fori_loop(0, NCH, chunk, 0)
    plsc.subcore_barrier()
    pltpu.sync_copy(acc_sh.at[pl.ds(s * nz, nz)], out_hbm.at[c, pl.ds(s * nz, nz)])


def _mlp(h, a0, a1, w1, b1, g1, be1, w2, b2, bg, bb, ep, relu_out):
    def body(h_ref, a0_ref, a1_ref, w1_ref, b1_ref, g1_ref, be1_ref, w2_ref,
             b2_ref, bg_ref, bb_ref, ep_ref, o_ref):
        z = (1.0 + ep_ref[0, 0]) * h_ref[...] + a0_ref[...] + a1_ref[...]
        z1 = jnp.dot(z, w1_ref[...], preferred_element_type=jnp.float32)
        z1 = z1 + b1_ref[...]
        m = jnp.mean(z1, axis=0, keepdims=True)
        z1 = z1 - m
        v = jnp.mean(z1 * z1, axis=0, keepdims=True)
        z1 = z1 * lax.rsqrt(v + 1e-5) * g1_ref[...] + be1_ref[...]
        z1 = jnp.maximum(z1, 0.0)
        z2 = jnp.dot(z1, w2_ref[...], preferred_element_type=jnp.float32)
        z2 = z2 + b2_ref[...]
        m2 = jnp.mean(z2, axis=0, keepdims=True)
        z2 = z2 - m2
        v2 = jnp.mean(z2 * z2, axis=0, keepdims=True)
        z2 = z2 * lax.rsqrt(v2 + 1e-5) * bg_ref[...] + bb_ref[...]
        if relu_out:
            z2 = jnp.maximum(z2, 0.0)
        o_ref[...] = z2

    return pl.pallas_call(
        body,
        out_shape=jax.ShapeDtypeStruct((N, D), jnp.float32),
    )(h, a0, a1, w1, b1.reshape(1, -1), g1.reshape(1, -1), be1.reshape(1, -1),
      w2, b2.reshape(1, -1), bg.reshape(1, -1), bb.reshape(1, -1),
      ep.reshape(1, 1))


def kernel(x, edge_index, edge_attr, edge_weight, atom_emb, bond_emb, W1, b1,
           g1, be1, W2, b2, eps, bn_g, bn_b):
    xi = x.astype(jnp.int32)
    idx_atom = (xi + jnp.arange(9, dtype=jnp.int32)[None, :] * 119).T
    padn = jnp.broadcast_to(
        (jnp.arange(NPAD - N, dtype=jnp.int32) % (9 * 119))[None, :],
        (9, NPAD - N))
    idx_atom = jnp.concatenate([idx_atom, padn], axis=1).reshape(9 * NPAD)
    atab = atom_emb.astype(jnp.float32).reshape(9 * 119, D)

    row = edge_index[0].astype(jnp.int32)
    col = edge_index[1].astype(jnp.int32)
    ea = edge_attr.astype(jnp.int32)
    comb = ea[:, 0] * 25 + ea[:, 1] * 5 + ea[:, 2]
    npe = EPAD - E
    pr = jnp.arange(npe, dtype=jnp.int32)
    row_p = jnp.concatenate([row, pr % N])
    col_p = jnp.concatenate([col, pr % N])
    comb_p = jnp.concatenate([comb, pr % 125])
    ew_p = jnp.concatenate(
        [edge_weight.astype(jnp.float32), jnp.zeros((npe,), jnp.float32)])

    ct = (bond_emb[:, 0][:, :, None, None, :]
          + bond_emb[:, 1][:, None, :, None, :]
          + bond_emb[:, 2][:, None, None, :, :]).reshape(NLAYER, 125, D)
    ct = jnp.concatenate(
        [ct, jnp.zeros((NLAYER, 3, D), jnp.float32)], axis=1)
    zer = jnp.zeros((NPAD, D), jnp.float32)

    h = _atom_kernel(atab, idx_atom)[:N]
    for l in range(NLAYER):
        agg = _agg_kernel(h, row_p, col_p, comb_p, ew_p, ct[l], zer)
        h = _mlp(h, agg[0, :N], agg[1, :N], W1[l], b1[l], g1[l], be1[l], W2[l], b2[l],
                 bn_g[l], bn_b[l], eps[l], relu_out=(l < NLAYER - 1))
    return h
